# baseline (speedup 1.0000x reference)
import functools

import jax
import jax.numpy as jnp
from jax import lax
from jax.experimental import pallas as pl
from jax.experimental.pallas import tpu as pltpu
from jax.experimental.pallas import tpu_sc as plsc

N_NODES = 10000
N_EDGES = 160000
D = 128
PW = 128
CH = 128
NCHUNK = N_EDGES // CH
NC, NS = 2, 16
NW = NC * NS
ROWS_A = 624
ROWS_TAIL = N_NODES - ROWS_A * NS


def _silu(x):
    return x * lax.logistic(x)


def _pre_project(V0, W1s, W1r):
    BN = 1000

    def body(v_ref, a_ref, b_ref, os_ref, or_ref):
        v = v_ref[...]
        os_ref[...] = jnp.dot(v, a_ref[...], preferred_element_type=jnp.float32)
        or_ref[...] = jnp.dot(v, b_ref[...], preferred_element_type=jnp.float32)

    return pl.pallas_call(
        body,
        grid=(N_NODES // BN,),
        in_specs=[
            pl.BlockSpec((BN, D), lambda i: (i, 0)),
            pl.BlockSpec((D, D), lambda i: (0, 0)),
            pl.BlockSpec((D, D), lambda i: (0, 0)),
        ],
        out_specs=[pl.BlockSpec((BN, D), lambda i: (i, 0))] * 2,
        out_shape=[jax.ShapeDtypeStruct((N_NODES, D), jnp.float32)] * 2,
    )(V0, W1s, W1r)


def _sc_gather(vws, vwr, sidx, ridx):
    mesh = plsc.VectorSubcoreMesh(core_axis_name="c", subcore_axis_name="s")

    @functools.partial(
        pl.kernel,
        out_type=(
            jax.ShapeDtypeStruct((N_EDGES, D), jnp.float32),
            jax.ShapeDtypeStruct((N_EDGES, D), jnp.float32),
        ),
        mesh=mesh,
        scratch_types=[
            pltpu.VMEM((CH,), jnp.int32),
            pltpu.VMEM((CH,), jnp.int32),
            pltpu.VMEM((CH, D), jnp.float32),
            pltpu.VMEM((CH, D), jnp.float32),
            pltpu.SemaphoreType.DMA,
            pltpu.SemaphoreType.DMA,
        ],
    )
    def k(vws_hbm, vwr_hbm, sidx_hbm, ridx_hbm, gs_hbm, gr_hbm,
          sidx_v, ridx_v, rows_s, rows_r, sem1, sem2):
        wid = lax.axis_index("s") * NC + lax.axis_index("c")
        trips = 39 + jnp.where(wid < NCHUNK - 39 * NW, 1, 0)

        def body(kk, carry):
            base = (wid + kk * NW) * CH
            cp1 = pltpu.async_copy(sidx_hbm.at[pl.ds(base, CH)], sidx_v, sem1)
            cp2 = pltpu.async_copy(ridx_hbm.at[pl.ds(base, CH)], ridx_v, sem2)
            cp1.wait()
            cp2.wait()
            cp3 = pltpu.async_copy(vws_hbm.at[sidx_v], rows_s, sem1)
            cp4 = pltpu.async_copy(vwr_hbm.at[ridx_v], rows_r, sem2)
            cp3.wait()
            cp4.wait()
            cp5 = pltpu.async_copy(rows_s, gs_hbm.at[pl.ds(base, CH)], sem1)
            cp6 = pltpu.async_copy(rows_r, gr_hbm.at[pl.ds(base, CH)], sem2)
            cp5.wait()
            cp6.wait()
            return carry

        lax.fori_loop(0, trips, body, 0)

    return k(vws, vwr, sidx, ridx)


def _edge_mlp(gs, gr, E0, We1c, be1, We2, be2, Ws1, bs1, Ws2, bs2,
              Wr1, br1, Wr2, br2, Was1, bas1, was2, bas2,
              War1, bar1, war2, bar2):
    BE = 2000

    def body(gs_ref, gr_ref, e_ref, w1c_ref, b1_ref, w2_ref, b2_ref,
             ws1_ref, bs1_ref, ws2_ref, bs2_ref,
             wr1_ref, br1_ref, wr2_ref, br2_ref,
             was1_ref, bas1_ref, was2_ref, bas2_ref,
             war1_ref, bar1_ref, war2_ref, bar2_ref,
             emb_ref, ps_ref, pr_ref):
        dot = lambda a, b: jnp.dot(a.astype(jnp.bfloat16),
                                   b.astype(jnp.bfloat16),
                                   preferred_element_type=jnp.float32)
        x = gs_ref[...] + gr_ref[...] + dot(e_ref[...], w1c_ref[...]) + b1_ref[...]
        emb = dot(_silu(x), w2_ref[...]) + b2_ref[...]
        emb_ref[...] = emb
        ms = dot(_silu(dot(emb, ws1_ref[...]) + bs1_ref[...]), ws2_ref[...]) + bs2_ref[...]
        mr = dot(_silu(dot(emb, wr1_ref[...]) + br1_ref[...]), wr2_ref[...]) + br2_ref[...]
        has = _silu(dot(emb, was1_ref[...]) + bas1_ref[...])
        har = _silu(dot(emb, war1_ref[...]) + bar1_ref[...])
        ls = jnp.clip(jnp.sum(has * was2_ref[...], axis=1) + bas2_ref[0, 0], -30.0, 30.0)
        lr = jnp.clip(jnp.sum(har * war2_ref[...], axis=1) + bar2_ref[0, 0], -30.0, 30.0)
        p = jnp.exp(ls)[:, None]
        q = jnp.exp(lr)[:, None]
        zpad = jnp.zeros((BE, PW - 65), jnp.float32)
        ps_ref[...] = jnp.concatenate([p * ms, p, zpad], axis=1)
        pr_ref[...] = jnp.concatenate([q * mr, q, zpad], axis=1)

    full = lambda shp: pl.BlockSpec(shp, lambda i: tuple(0 for _ in shp))
    return pl.pallas_call(
        body,
        grid=(N_EDGES // BE,),
        in_specs=[
            pl.BlockSpec((BE, D), lambda i: (i, 0)),
            pl.BlockSpec((BE, D), lambda i: (i, 0)),
            pl.BlockSpec((BE, D), lambda i: (i, 0)),
            full((D, D)), full((1, D)), full((D, D)), full((1, D)),
            full((D, D)), full((1, D)), full((D, 64)), full((1, 64)),
            full((D, D)), full((1, D)), full((D, 64)), full((1, 64)),
            full((D, 32)), full((1, 32)), full((1, 32)), full((1, 1)),
            full((D, 32)), full((1, 32)), full((1, 32)), full((1, 1)),
        ],
        out_specs=[
            pl.BlockSpec((BE, D), lambda i: (i, 0)),
            pl.BlockSpec((BE, PW), lambda i: (i, 0)),
            pl.BlockSpec((BE, PW), lambda i: (i, 0)),
        ],
        out_shape=[
            jax.ShapeDtypeStruct((N_EDGES, D), jnp.float32),
            jax.ShapeDtypeStruct((N_EDGES, PW), jnp.float32),
            jax.ShapeDtypeStruct((N_EDGES, PW), jnp.float32),
        ],
    )(gs, gr, E0, We1c, be1, We2, be2, Ws1, bs1, Ws2, bs2,
      Wr1, br1, Wr2, br2, Was1, bas1, was2, bas2, War1, bar1, war2, bar2)


def _sc_scatter(pay_s, pay_r, sidx, ridx, zrows):
    mesh = plsc.VectorSubcoreMesh(core_axis_name="c", subcore_axis_name="s")

    @functools.partial(
        pl.kernel,
        out_type=jax.ShapeDtypeStruct((NC, N_NODES, PW), jnp.float32),
        mesh=mesh,
        scratch_types=[
            pltpu.VMEM((CH,), jnp.int32),
            pltpu.VMEM((CH, PW), jnp.float32),
            pltpu.VMEM_SHARED((N_NODES, PW), jnp.float32),
            pltpu.SemaphoreType.DMA,
            pltpu.SemaphoreType.DMA,
        ],
    )
    def k(ps_hbm, pr_hbm, sidx_hbm, ridx_hbm, z_hbm, out_hbm,
          idx_v, pay_v, acc, sem1, sem2):
        cid = lax.axis_index("c")
        sid = lax.axis_index("s")
        r0 = pl.multiple_of(sid * ROWS_A, 8)
        t0 = ROWS_A * NS
        pltpu.sync_copy(z_hbm, acc.at[pl.ds(r0, ROWS_A)])

        @pl.when(sid == NS - 1)
        def _():
            pltpu.sync_copy(z_hbm.at[pl.ds(0, ROWS_TAIL)],
                            acc.at[pl.ds(t0, ROWS_TAIL)])

        plsc.subcore_barrier()

        trips = 78 + jnp.where(sid < NCHUNK - 78 * NS, 1, 0)

        def mk_body(idx_hbm, p_hbm):
            def body(kk, carry):
                base = (sid + kk * NS) * CH
                cp1 = pltpu.async_copy(idx_hbm.at[pl.ds(base, CH)], idx_v, sem1)
                cp2 = pltpu.async_copy(p_hbm.at[pl.ds(base, CH)], pay_v, sem2)
                cp1.wait()
                cp2.wait()
                pltpu.sync_copy(pay_v, acc.at[idx_v], add=True)
                return carry
            return body

        @pl.when(cid == 0)
        def _():
            lax.fori_loop(0, trips, mk_body(sidx_hbm, ps_hbm), 0)

        @pl.when(cid == 1)
        def _():
            lax.fori_loop(0, trips, mk_body(ridx_hbm, pr_hbm), 0)

        plsc.subcore_barrier()
        pltpu.sync_copy(acc.at[pl.ds(r0, ROWS_A)],
                        out_hbm.at[cid, pl.ds(r0, ROWS_A)])

        @pl.when(sid == NS - 1)
        def _():
            pltpu.sync_copy(acc.at[pl.ds(t0, ROWS_TAIL)],
                            out_hbm.at[cid, pl.ds(t0, ROWS_TAIL)])

    return k(pay_s, pay_r, sidx, ridx, zrows)


def _node_mlp(V0, parts, Wn1, bn1, Wn2, bn2):
    BN = 1000

    def body(v_ref, parts_ref, wn1_ref, bn1_ref, wn2_ref, bn2_ref, o_ref):
        pr = parts_ref[...]
        ps = pr[0]
        pr_ = pr[1]
        agg0 = ps[:, 0:64] / (ps[:, 64:65] + 1e-30)
        agg1 = pr_[:, 0:64] / (pr_[:, 64:65] + 1e-30)
        ni = jnp.concatenate([v_ref[...], agg0, agg1], axis=1)
        h = _silu(jnp.dot(ni, wn1_ref[...], preferred_element_type=jnp.float32)
                  + bn1_ref[...])
        o_ref[...] = (jnp.dot(h, wn2_ref[...], preferred_element_type=jnp.float32)
                      + bn2_ref[...])

    return pl.pallas_call(
        body,
        grid=(N_NODES // BN,),
        in_specs=[
            pl.BlockSpec((BN, D), lambda i: (i, 0)),
            pl.BlockSpec((NC, BN, PW), lambda i: (0, i, 0)),
            pl.BlockSpec((2 * D, D), lambda i: (0, 0)),
            pl.BlockSpec((1, D), lambda i: (0, 0)),
            pl.BlockSpec((D, D), lambda i: (0, 0)),
            pl.BlockSpec((1, D), lambda i: (0, 0)),
        ],
        out_specs=pl.BlockSpec((BN, D), lambda i: (i, 0)),
        out_shape=jax.ShapeDtypeStruct((N_NODES, D), jnp.float32),
    )(V0, parts, Wn1, bn1, Wn2, bn2)


def kernel(V, E, edges, We1, be1, We2, be2, Ws1, bs1, Ws2, bs2,
           Wr1, br1, Wr2, br2, Was1, bas1, Was2, bas2,
           War1, bar1, War2, bar2, Wn1, bn1, Wn2, bn2):
    V0 = V[0]
    E0 = E[0]
    sidx = edges[0, :, 0]
    ridx = edges[0, :, 1]

    vws, vwr = _pre_project(V0, We1[0:D], We1[D:2 * D])
    gs, gr = _sc_gather(vws, vwr, sidx, ridx)
    emb, pay_s, pay_r = _edge_mlp(
        gs, gr, E0, We1[2 * D:], be1.reshape(1, D), We2, be2.reshape(1, D),
        Ws1, bs1.reshape(1, D), Ws2, bs2.reshape(1, 64),
        Wr1, br1.reshape(1, D), Wr2, br2.reshape(1, 64),
        Was1, bas1.reshape(1, 32), Was2.reshape(1, 32), bas2.reshape(1, 1),
        War1, bar1.reshape(1, 32), War2.reshape(1, 32), bar2.reshape(1, 1))
    zrows = jnp.zeros((ROWS_A, PW), jnp.float32)
    parts = _sc_scatter(pay_s, pay_r, sidx, ridx, zrows)
    node_emb = _node_mlp(V0, parts, Wn1, bn1.reshape(1, D), Wn2,
                         bn2.reshape(1, D))
    return node_emb[None], emb[None]


# speedup vs baseline: 1.0584x; 1.0584x over previous
import functools

import jax
import jax.numpy as jnp
from jax import lax
from jax.experimental import pallas as pl
from jax.experimental.pallas import tpu as pltpu
from jax.experimental.pallas import tpu_sc as plsc

N_NODES = 10000
N_EDGES = 160000
D = 128
PW = 128
CH = 128
NCHUNK = N_EDGES // CH
NC, NS = 2, 16
NW = NC * NS
ROWS_A = 624
ROWS_TAIL = N_NODES - ROWS_A * NS


def _silu(x):
    return x * lax.logistic(x)


def _pre_project(V0, W1s, W1r):
    BN = 1000

    def body(v_ref, a_ref, b_ref, os_ref, or_ref):
        v = v_ref[...]
        os_ref[...] = jnp.dot(v, a_ref[...], preferred_element_type=jnp.float32)
        or_ref[...] = jnp.dot(v, b_ref[...], preferred_element_type=jnp.float32)

    return pl.pallas_call(
        body,
        grid=(N_NODES // BN,),
        in_specs=[
            pl.BlockSpec((BN, D), lambda i: (i, 0)),
            pl.BlockSpec((D, D), lambda i: (0, 0)),
            pl.BlockSpec((D, D), lambda i: (0, 0)),
        ],
        out_specs=[pl.BlockSpec((BN, D), lambda i: (i, 0))] * 2,
        out_shape=[jax.ShapeDtypeStruct((N_NODES, D), jnp.float32)] * 2,
    )(V0, W1s, W1r)


def _sc_gather(vws, vwr, sidx, ridx):
    mesh = plsc.VectorSubcoreMesh(core_axis_name="c", subcore_axis_name="s")

    @functools.partial(
        pl.kernel,
        out_type=(
            jax.ShapeDtypeStruct((N_EDGES, D), jnp.float32),
            jax.ShapeDtypeStruct((N_EDGES, D), jnp.float32),
        ),
        mesh=mesh,
        scratch_types=[
            pltpu.VMEM((CH,), jnp.int32),
            pltpu.VMEM((CH,), jnp.int32),
            pltpu.VMEM((CH, D), jnp.float32),
            pltpu.VMEM((CH, D), jnp.float32),
            pltpu.SemaphoreType.DMA,
            pltpu.SemaphoreType.DMA,
        ],
    )
    def k(vws_hbm, vwr_hbm, sidx_hbm, ridx_hbm, gs_hbm, gr_hbm,
          sidx_v, ridx_v, rows_s, rows_r, sem1, sem2):
        wid = lax.axis_index("s") * NC + lax.axis_index("c")
        trips = 39 + jnp.where(wid < NCHUNK - 39 * NW, 1, 0)

        def body(kk, carry):
            base = (wid + kk * NW) * CH
            cp1 = pltpu.async_copy(sidx_hbm.at[pl.ds(base, CH)], sidx_v, sem1)
            cp2 = pltpu.async_copy(ridx_hbm.at[pl.ds(base, CH)], ridx_v, sem2)
            cp1.wait()
            cp2.wait()
            cp3 = pltpu.async_copy(vws_hbm.at[sidx_v], rows_s, sem1)
            cp4 = pltpu.async_copy(vwr_hbm.at[ridx_v], rows_r, sem2)
            cp3.wait()
            cp4.wait()
            cp5 = pltpu.async_copy(rows_s, gs_hbm.at[pl.ds(base, CH)], sem1)
            cp6 = pltpu.async_copy(rows_r, gr_hbm.at[pl.ds(base, CH)], sem2)
            cp5.wait()
            cp6.wait()
            return carry

        lax.fori_loop(0, trips, body, 0)

    return k(vws, vwr, sidx, ridx)


def _edge_mlp(gs, gr, E0, We1c, be1, We2, be2, Ws1, bs1, Ws2, bs2,
              Wr1, br1, Wr2, br2, Was1, bas1, was2, bas2,
              War1, bar1, war2, bar2):
    BE = 2000

    def body(gs_ref, gr_ref, e_ref, w1c_ref, b1_ref, w2_ref, b2_ref,
             ws1_ref, bs1_ref, ws2_ref, bs2_ref,
             wr1_ref, br1_ref, wr2_ref, br2_ref,
             was1_ref, bas1_ref, was2_ref, bas2_ref,
             war1_ref, bar1_ref, war2_ref, bar2_ref,
             emb_ref, ps_ref, pr_ref):
        dot = lambda a, b: jnp.dot(a, b, preferred_element_type=jnp.float32)
        x = gs_ref[...] + gr_ref[...] + dot(e_ref[...], w1c_ref[...]) + b1_ref[...]
        emb = dot(_silu(x), w2_ref[...]) + b2_ref[...]
        emb_ref[...] = emb
        ms = dot(_silu(dot(emb, ws1_ref[...]) + bs1_ref[...]), ws2_ref[...]) + bs2_ref[...]
        mr = dot(_silu(dot(emb, wr1_ref[...]) + br1_ref[...]), wr2_ref[...]) + br2_ref[...]
        has = _silu(dot(emb, was1_ref[...]) + bas1_ref[...])
        har = _silu(dot(emb, war1_ref[...]) + bar1_ref[...])
        ls = jnp.clip(jnp.sum(has * was2_ref[...], axis=1) + bas2_ref[0, 0], -30.0, 30.0)
        lr = jnp.clip(jnp.sum(har * war2_ref[...], axis=1) + bar2_ref[0, 0], -30.0, 30.0)
        p = jnp.exp(ls)[:, None]
        q = jnp.exp(lr)[:, None]
        zpad = jnp.zeros((BE, PW - 65), jnp.float32)
        ps_ref[...] = jnp.concatenate([p * ms, p, zpad], axis=1)
        pr_ref[...] = jnp.concatenate([q * mr, q, zpad], axis=1)

    full = lambda shp: pl.BlockSpec(shp, lambda i: tuple(0 for _ in shp))
    return pl.pallas_call(
        body,
        grid=(N_EDGES // BE,),
        in_specs=[
            pl.BlockSpec((BE, D), lambda i: (i, 0)),
            pl.BlockSpec((BE, D), lambda i: (i, 0)),
            pl.BlockSpec((BE, D), lambda i: (i, 0)),
            full((D, D)), full((1, D)), full((D, D)), full((1, D)),
            full((D, D)), full((1, D)), full((D, 64)), full((1, 64)),
            full((D, D)), full((1, D)), full((D, 64)), full((1, 64)),
            full((D, 32)), full((1, 32)), full((1, 32)), full((1, 1)),
            full((D, 32)), full((1, 32)), full((1, 32)), full((1, 1)),
        ],
        out_specs=[
            pl.BlockSpec((BE, D), lambda i: (i, 0)),
            pl.BlockSpec((BE, PW), lambda i: (i, 0)),
            pl.BlockSpec((BE, PW), lambda i: (i, 0)),
        ],
        out_shape=[
            jax.ShapeDtypeStruct((N_EDGES, D), jnp.float32),
            jax.ShapeDtypeStruct((N_EDGES, PW), jnp.float32),
            jax.ShapeDtypeStruct((N_EDGES, PW), jnp.float32),
        ],
    )(gs, gr, E0, We1c, be1, We2, be2, Ws1, bs1, Ws2, bs2,
      Wr1, br1, Wr2, br2, Was1, bas1, was2, bas2, War1, bar1, war2, bar2)


def _sc_scatter(pay_s, pay_r, sidx, ridx, zrows):
    mesh = plsc.VectorSubcoreMesh(core_axis_name="c", subcore_axis_name="s")

    @functools.partial(
        pl.kernel,
        out_type=jax.ShapeDtypeStruct((NC, N_NODES, PW), jnp.float32),
        mesh=mesh,
        scratch_types=[
            pltpu.VMEM((CH,), jnp.int32),
            pltpu.VMEM((CH, PW), jnp.float32),
            pltpu.VMEM_SHARED((N_NODES, PW), jnp.float32),
            pltpu.SemaphoreType.DMA,
            pltpu.SemaphoreType.DMA,
        ],
    )
    def k(ps_hbm, pr_hbm, sidx_hbm, ridx_hbm, z_hbm, out_hbm,
          idx_v, pay_v, acc, sem1, sem2):
        cid = lax.axis_index("c")
        sid = lax.axis_index("s")
        r0 = pl.multiple_of(sid * ROWS_A, 8)
        t0 = ROWS_A * NS
        pltpu.sync_copy(z_hbm, acc.at[pl.ds(r0, ROWS_A)])

        @pl.when(sid == NS - 1)
        def _():
            pltpu.sync_copy(z_hbm.at[pl.ds(0, ROWS_TAIL)],
                            acc.at[pl.ds(t0, ROWS_TAIL)])

        plsc.subcore_barrier()

        trips = 78 + jnp.where(sid < NCHUNK - 78 * NS, 1, 0)

        def mk_body(idx_hbm, p_hbm):
            def body(kk, carry):
                base = (sid + kk * NS) * CH
                cp1 = pltpu.async_copy(idx_hbm.at[pl.ds(base, CH)], idx_v, sem1)
                cp2 = pltpu.async_copy(p_hbm.at[pl.ds(base, CH)], pay_v, sem2)
                cp1.wait()
                cp2.wait()
                pltpu.sync_copy(pay_v, acc.at[idx_v], add=True)
                return carry
            return body

        @pl.when(cid == 0)
        def _():
            lax.fori_loop(0, trips, mk_body(sidx_hbm, ps_hbm), 0)

        @pl.when(cid == 1)
        def _():
            lax.fori_loop(0, trips, mk_body(ridx_hbm, pr_hbm), 0)

        plsc.subcore_barrier()
        pltpu.sync_copy(acc.at[pl.ds(r0, ROWS_A)],
                        out_hbm.at[cid, pl.ds(r0, ROWS_A)])

        @pl.when(sid == NS - 1)
        def _():
            pltpu.sync_copy(acc.at[pl.ds(t0, ROWS_TAIL)],
                            out_hbm.at[cid, pl.ds(t0, ROWS_TAIL)])

    return k(pay_s, pay_r, sidx, ridx, zrows)


def _node_mlp(V0, parts, Wn1, bn1, Wn2, bn2):
    BN = 1000

    def body(v_ref, parts_ref, wn1_ref, bn1_ref, wn2_ref, bn2_ref, o_ref):
        pr = parts_ref[...]
        ps = pr[0]
        pr_ = pr[1]
        agg0 = ps[:, 0:64] / (ps[:, 64:65] + 1e-30)
        agg1 = pr_[:, 0:64] / (pr_[:, 64:65] + 1e-30)
        ni = jnp.concatenate([v_ref[...], agg0, agg1], axis=1)
        h = _silu(jnp.dot(ni, wn1_ref[...], preferred_element_type=jnp.float32)
                  + bn1_ref[...])
        o_ref[...] = (jnp.dot(h, wn2_ref[...], preferred_element_type=jnp.float32)
                      + bn2_ref[...])

    return pl.pallas_call(
        body,
        grid=(N_NODES // BN,),
        in_specs=[
            pl.BlockSpec((BN, D), lambda i: (i, 0)),
            pl.BlockSpec((NC, BN, PW), lambda i: (0, i, 0)),
            pl.BlockSpec((2 * D, D), lambda i: (0, 0)),
            pl.BlockSpec((1, D), lambda i: (0, 0)),
            pl.BlockSpec((D, D), lambda i: (0, 0)),
            pl.BlockSpec((1, D), lambda i: (0, 0)),
        ],
        out_specs=pl.BlockSpec((BN, D), lambda i: (i, 0)),
        out_shape=jax.ShapeDtypeStruct((N_NODES, D), jnp.float32),
    )(V0, parts, Wn1, bn1, Wn2, bn2)


def kernel(V, E, edges, We1, be1, We2, be2, Ws1, bs1, Ws2, bs2,
           Wr1, br1, Wr2, br2, Was1, bas1, Was2, bas2,
           War1, bar1, War2, bar2, Wn1, bn1, Wn2, bn2):
    V0 = V[0]
    E0 = E[0]
    sidx = edges[0, :, 0]
    ridx = edges[0, :, 1]

    vws, vwr = _pre_project(V0, We1[0:D], We1[D:2 * D])
    gs, gr = _sc_gather(vws, vwr, sidx, ridx)
    emb, pay_s, pay_r = _edge_mlp(
        gs, gr, E0, We1[2 * D:], be1.reshape(1, D), We2, be2.reshape(1, D),
        Ws1, bs1.reshape(1, D), Ws2, bs2.reshape(1, 64),
        Wr1, br1.reshape(1, D), Wr2, br2.reshape(1, 64),
        Was1, bas1.reshape(1, 32), Was2.reshape(1, 32), bas2.reshape(1, 1),
        War1, bar1.reshape(1, 32), War2.reshape(1, 32), bar2.reshape(1, 1))
    zrows = jnp.zeros((ROWS_A, PW), jnp.float32)
    parts = _sc_scatter(pay_s, pay_r, sidx, ridx, zrows)
    node_emb = _node_mlp(V0, parts, Wn1, bn1.reshape(1, D), Wn2,
                         bn2.reshape(1, D))
    return node_emb[None], emb[None]


# speedup vs baseline: 1.1079x; 1.0468x over previous
import functools

import jax
import jax.numpy as jnp
from jax import lax
from jax.experimental import pallas as pl
from jax.experimental.pallas import tpu as pltpu
from jax.experimental.pallas import tpu_sc as plsc

N_NODES = 10000
N_EDGES = 160000
D = 128
PW = 128
CH = 128
NCHUNK = N_EDGES // CH
NC, NS = 2, 16
NW = NC * NS
ROWS_A = 624
ROWS_TAIL = N_NODES - ROWS_A * NS


def _silu(x):
    return x * lax.logistic(x)


def _pre_project(V0, W1s, W1r):
    BN = 1000

    def body(v_ref, a_ref, b_ref, os_ref, or_ref):
        v = v_ref[...]
        os_ref[...] = jnp.dot(v, a_ref[...], preferred_element_type=jnp.float32)
        or_ref[...] = jnp.dot(v, b_ref[...], preferred_element_type=jnp.float32)

    return pl.pallas_call(
        body,
        grid=(N_NODES // BN,),
        in_specs=[
            pl.BlockSpec((BN, D), lambda i: (i, 0)),
            pl.BlockSpec((D, D), lambda i: (0, 0)),
            pl.BlockSpec((D, D), lambda i: (0, 0)),
        ],
        out_specs=[pl.BlockSpec((BN, D), lambda i: (i, 0))] * 2,
        out_shape=[jax.ShapeDtypeStruct((N_NODES, D), jnp.float32)] * 2,
    )(V0, W1s, W1r)


def _sc_gather(vws, vwr, sidx, ridx):
    mesh = plsc.VectorSubcoreMesh(core_axis_name="c", subcore_axis_name="s")

    @functools.partial(
        pl.kernel,
        out_type=(
            jax.ShapeDtypeStruct((N_EDGES, D), jnp.float32),
            jax.ShapeDtypeStruct((N_EDGES, D), jnp.float32),
        ),
        mesh=mesh,
        scratch_types=[
            pltpu.VMEM((CH,), jnp.int32),
            pltpu.VMEM((CH,), jnp.int32),
            pltpu.VMEM((CH, D), jnp.float32),
            pltpu.VMEM((CH, D), jnp.float32),
            pltpu.SemaphoreType.DMA,
            pltpu.SemaphoreType.DMA,
        ],
    )
    def k(vws_hbm, vwr_hbm, sidx_hbm, ridx_hbm, gs_hbm, gr_hbm,
          sidx_v, ridx_v, rows_s, rows_r, sem1, sem2):
        wid = lax.axis_index("s") * NC + lax.axis_index("c")
        trips = 39 + jnp.where(wid < NCHUNK - 39 * NW, 1, 0)

        def body(kk, carry):
            base = (wid + kk * NW) * CH
            cp1 = pltpu.async_copy(sidx_hbm.at[pl.ds(base, CH)], sidx_v, sem1)
            cp2 = pltpu.async_copy(ridx_hbm.at[pl.ds(base, CH)], ridx_v, sem2)
            cp1.wait()
            cp2.wait()
            cp3 = pltpu.async_copy(vws_hbm.at[sidx_v], rows_s, sem1)
            cp4 = pltpu.async_copy(vwr_hbm.at[ridx_v], rows_r, sem2)
            cp3.wait()
            cp4.wait()
            cp5 = pltpu.async_copy(rows_s, gs_hbm.at[pl.ds(base, CH)], sem1)
            cp6 = pltpu.async_copy(rows_r, gr_hbm.at[pl.ds(base, CH)], sem2)
            cp5.wait()
            cp6.wait()
            return carry

        lax.fori_loop(0, trips, body, 0)

    return k(vws, vwr, sidx, ridx)


def _edge_mlp(gs, gr, E0, We1c, be1, We2, be2, Ws1, bs1, Ws2, bs2,
              Wr1, br1, Wr2, br2, Was1, bas1, was2, bas2,
              War1, bar1, war2, bar2):
    BE = 4000

    def body(gs_ref, gr_ref, e_ref, w1c_ref, b1_ref, w2_ref, b2_ref,
             ws1_ref, bs1_ref, ws2_ref, bs2_ref,
             wr1_ref, br1_ref, wr2_ref, br2_ref,
             was1_ref, bas1_ref, was2_ref, bas2_ref,
             war1_ref, bar1_ref, war2_ref, bar2_ref,
             emb_ref, ps_ref, pr_ref):
        dot = lambda a, b: jnp.dot(a, b, preferred_element_type=jnp.float32)
        x = gs_ref[...] + gr_ref[...] + dot(e_ref[...], w1c_ref[...]) + b1_ref[...]
        emb = dot(_silu(x), w2_ref[...]) + b2_ref[...]
        emb_ref[...] = emb
        ms = dot(_silu(dot(emb, ws1_ref[...]) + bs1_ref[...]), ws2_ref[...]) + bs2_ref[...]
        mr = dot(_silu(dot(emb, wr1_ref[...]) + br1_ref[...]), wr2_ref[...]) + br2_ref[...]
        has = _silu(dot(emb, was1_ref[...]) + bas1_ref[...])
        har = _silu(dot(emb, war1_ref[...]) + bar1_ref[...])
        ls = jnp.clip(jnp.sum(has * was2_ref[...], axis=1) + bas2_ref[0, 0], -30.0, 30.0)
        lr = jnp.clip(jnp.sum(har * war2_ref[...], axis=1) + bar2_ref[0, 0], -30.0, 30.0)
        p = jnp.exp(ls)[:, None]
        q = jnp.exp(lr)[:, None]
        zpad = jnp.zeros((BE, PW - 65), jnp.float32)
        ps_ref[...] = jnp.concatenate([p * ms, p, zpad], axis=1)
        pr_ref[...] = jnp.concatenate([q * mr, q, zpad], axis=1)

    full = lambda shp: pl.BlockSpec(shp, lambda i: tuple(0 for _ in shp))
    return pl.pallas_call(
        body,
        grid=(N_EDGES // BE,),
        in_specs=[
            pl.BlockSpec((BE, D), lambda i: (i, 0)),
            pl.BlockSpec((BE, D), lambda i: (i, 0)),
            pl.BlockSpec((BE, D), lambda i: (i, 0)),
            full((D, D)), full((1, D)), full((D, D)), full((1, D)),
            full((D, D)), full((1, D)), full((D, 64)), full((1, 64)),
            full((D, D)), full((1, D)), full((D, 64)), full((1, 64)),
            full((D, 32)), full((1, 32)), full((1, 32)), full((1, 1)),
            full((D, 32)), full((1, 32)), full((1, 32)), full((1, 1)),
        ],
        out_specs=[
            pl.BlockSpec((BE, D), lambda i: (i, 0)),
            pl.BlockSpec((BE, PW), lambda i: (i, 0)),
            pl.BlockSpec((BE, PW), lambda i: (i, 0)),
        ],
        out_shape=[
            jax.ShapeDtypeStruct((N_EDGES, D), jnp.float32),
            jax.ShapeDtypeStruct((N_EDGES, PW), jnp.float32),
            jax.ShapeDtypeStruct((N_EDGES, PW), jnp.float32),
        ],
    )(gs, gr, E0, We1c, be1, We2, be2, Ws1, bs1, Ws2, bs2,
      Wr1, br1, Wr2, br2, Was1, bas1, was2, bas2, War1, bar1, war2, bar2)


def _sc_scatter(pay_s, pay_r, sidx, ridx, zrows):
    mesh = plsc.VectorSubcoreMesh(core_axis_name="c", subcore_axis_name="s")

    @functools.partial(
        pl.kernel,
        out_type=jax.ShapeDtypeStruct((NC, N_NODES, PW), jnp.float32),
        mesh=mesh,
        scratch_types=[
            pltpu.VMEM((CH,), jnp.int32),
            pltpu.VMEM((CH, PW), jnp.float32),
            pltpu.VMEM_SHARED((N_NODES, PW), jnp.float32),
            pltpu.SemaphoreType.DMA,
            pltpu.SemaphoreType.DMA,
        ],
    )
    def k(ps_hbm, pr_hbm, sidx_hbm, ridx_hbm, z_hbm, out_hbm,
          idx_v, pay_v, acc, sem1, sem2):
        cid = lax.axis_index("c")
        sid = lax.axis_index("s")
        r0 = pl.multiple_of(sid * ROWS_A, 8)
        t0 = ROWS_A * NS
        pltpu.sync_copy(z_hbm, acc.at[pl.ds(r0, ROWS_A)])

        @pl.when(sid == NS - 1)
        def _():
            pltpu.sync_copy(z_hbm.at[pl.ds(0, ROWS_TAIL)],
                            acc.at[pl.ds(t0, ROWS_TAIL)])

        plsc.subcore_barrier()

        trips = 78 + jnp.where(sid < NCHUNK - 78 * NS, 1, 0)

        def mk_body(idx_hbm, p_hbm):
            def body(kk, carry):
                base = (sid + kk * NS) * CH
                cp1 = pltpu.async_copy(idx_hbm.at[pl.ds(base, CH)], idx_v, sem1)
                cp2 = pltpu.async_copy(p_hbm.at[pl.ds(base, CH)], pay_v, sem2)
                cp1.wait()
                cp2.wait()
                pltpu.sync_copy(pay_v, acc.at[idx_v], add=True)
                return carry
            return body

        @pl.when(cid == 0)
        def _():
            lax.fori_loop(0, trips, mk_body(sidx_hbm, ps_hbm), 0)

        @pl.when(cid == 1)
        def _():
            lax.fori_loop(0, trips, mk_body(ridx_hbm, pr_hbm), 0)

        plsc.subcore_barrier()
        pltpu.sync_copy(acc.at[pl.ds(r0, ROWS_A)],
                        out_hbm.at[cid, pl.ds(r0, ROWS_A)])

        @pl.when(sid == NS - 1)
        def _():
            pltpu.sync_copy(acc.at[pl.ds(t0, ROWS_TAIL)],
                            out_hbm.at[cid, pl.ds(t0, ROWS_TAIL)])

    return k(pay_s, pay_r, sidx, ridx, zrows)


def _node_mlp(V0, parts, Wn1, bn1, Wn2, bn2):
    BN = 1000

    def body(v_ref, parts_ref, wn1_ref, bn1_ref, wn2_ref, bn2_ref, o_ref):
        pr = parts_ref[...]
        ps = pr[0]
        pr_ = pr[1]
        agg0 = ps[:, 0:64] / (ps[:, 64:65] + 1e-30)
        agg1 = pr_[:, 0:64] / (pr_[:, 64:65] + 1e-30)
        ni = jnp.concatenate([v_ref[...], agg0, agg1], axis=1)
        h = _silu(jnp.dot(ni, wn1_ref[...], preferred_element_type=jnp.float32)
                  + bn1_ref[...])
        o_ref[...] = (jnp.dot(h, wn2_ref[...], preferred_element_type=jnp.float32)
                      + bn2_ref[...])

    return pl.pallas_call(
        body,
        grid=(N_NODES // BN,),
        in_specs=[
            pl.BlockSpec((BN, D), lambda i: (i, 0)),
            pl.BlockSpec((NC, BN, PW), lambda i: (0, i, 0)),
            pl.BlockSpec((2 * D, D), lambda i: (0, 0)),
            pl.BlockSpec((1, D), lambda i: (0, 0)),
            pl.BlockSpec((D, D), lambda i: (0, 0)),
            pl.BlockSpec((1, D), lambda i: (0, 0)),
        ],
        out_specs=pl.BlockSpec((BN, D), lambda i: (i, 0)),
        out_shape=jax.ShapeDtypeStruct((N_NODES, D), jnp.float32),
    )(V0, parts, Wn1, bn1, Wn2, bn2)


def kernel(V, E, edges, We1, be1, We2, be2, Ws1, bs1, Ws2, bs2,
           Wr1, br1, Wr2, br2, Was1, bas1, Was2, bas2,
           War1, bar1, War2, bar2, Wn1, bn1, Wn2, bn2):
    V0 = V[0]
    E0 = E[0]
    sidx = edges[0, :, 0]
    ridx = edges[0, :, 1]

    vws, vwr = _pre_project(V0, We1[0:D], We1[D:2 * D])
    gs, gr = _sc_gather(vws, vwr, sidx, ridx)
    emb, pay_s, pay_r = _edge_mlp(
        gs, gr, E0, We1[2 * D:], be1.reshape(1, D), We2, be2.reshape(1, D),
        Ws1, bs1.reshape(1, D), Ws2, bs2.reshape(1, 64),
        Wr1, br1.reshape(1, D), Wr2, br2.reshape(1, 64),
        Was1, bas1.reshape(1, 32), Was2.reshape(1, 32), bas2.reshape(1, 1),
        War1, bar1.reshape(1, 32), War2.reshape(1, 32), bar2.reshape(1, 1))
    zrows = jnp.zeros((ROWS_A, PW), jnp.float32)
    parts = _sc_scatter(pay_s, pay_r, sidx, ridx, zrows)
    node_emb = _node_mlp(V0, parts, Wn1, bn1.reshape(1, D), Wn2,
                         bn2.reshape(1, D))
    return node_emb[None], emb[None]


# speedup vs baseline: 1.1305x; 1.0204x over previous
import functools

import jax
import jax.numpy as jnp
from jax import lax
from jax.experimental import pallas as pl
from jax.experimental.pallas import tpu as pltpu
from jax.experimental.pallas import tpu_sc as plsc

N_NODES = 10000
N_EDGES = 160000
D = 128
PW = 128
CH = 128
NCHUNK = N_EDGES // CH
NC, NS = 2, 16
NW = NC * NS
ROWS_A = 624
ROWS_TAIL = N_NODES - ROWS_A * NS


def _silu(x):
    return x * lax.logistic(x)


def _pre_project(V0, W1s, W1r):
    BN = 1000

    def body(v_ref, a_ref, b_ref, os_ref, or_ref):
        v = v_ref[...]
        os_ref[...] = jnp.dot(v, a_ref[...], preferred_element_type=jnp.float32)
        or_ref[...] = jnp.dot(v, b_ref[...], preferred_element_type=jnp.float32)

    return pl.pallas_call(
        body,
        grid=(N_NODES // BN,),
        in_specs=[
            pl.BlockSpec((BN, D), lambda i: (i, 0)),
            pl.BlockSpec((D, D), lambda i: (0, 0)),
            pl.BlockSpec((D, D), lambda i: (0, 0)),
        ],
        out_specs=[pl.BlockSpec((BN, D), lambda i: (i, 0))] * 2,
        out_shape=[jax.ShapeDtypeStruct((N_NODES, D), jnp.float32)] * 2,
    )(V0, W1s, W1r)


def _sc_gather(vws, vwr, sidx, ridx):
    mesh = plsc.VectorSubcoreMesh(core_axis_name="c", subcore_axis_name="s")

    @functools.partial(
        pl.kernel,
        out_type=(
            jax.ShapeDtypeStruct((N_EDGES, D), jnp.float32),
            jax.ShapeDtypeStruct((N_EDGES, D), jnp.float32),
        ),
        mesh=mesh,
        scratch_types=[
            pltpu.VMEM((CH,), jnp.int32),
            pltpu.VMEM((CH,), jnp.int32),
            pltpu.VMEM((CH, D), jnp.float32),
            pltpu.VMEM((CH, D), jnp.float32),
            pltpu.SemaphoreType.DMA,
            pltpu.SemaphoreType.DMA,
        ],
    )
    def k(vws_hbm, vwr_hbm, sidx_hbm, ridx_hbm, gs_hbm, gr_hbm,
          sidx_v, ridx_v, rows_s, rows_r, sem1, sem2):
        wid = lax.axis_index("s") * NC + lax.axis_index("c")
        trips = 39 + jnp.where(wid < NCHUNK - 39 * NW, 1, 0)

        def body(kk, carry):
            base = (wid + kk * NW) * CH
            cp1 = pltpu.async_copy(sidx_hbm.at[pl.ds(base, CH)], sidx_v, sem1)
            cp2 = pltpu.async_copy(ridx_hbm.at[pl.ds(base, CH)], ridx_v, sem2)
            cp1.wait()
            cp2.wait()
            cp3 = pltpu.async_copy(vws_hbm.at[sidx_v], rows_s, sem1)
            cp4 = pltpu.async_copy(vwr_hbm.at[ridx_v], rows_r, sem2)
            cp3.wait()
            cp4.wait()
            cp5 = pltpu.async_copy(rows_s, gs_hbm.at[pl.ds(base, CH)], sem1)
            cp6 = pltpu.async_copy(rows_r, gr_hbm.at[pl.ds(base, CH)], sem2)
            cp5.wait()
            cp6.wait()
            return carry

        lax.fori_loop(0, trips, body, 0)

    return k(vws, vwr, sidx, ridx)


def _edge_mlp(gs, gr, E0, We1c, be1, We2, be2, Ws1, bs1, Ws2, bs2,
              Wr1, br1, Wr2, br2, Was1, bas1, was2, bas2,
              War1, bar1, war2, bar2):
    BE = 8000

    def body(gs_ref, gr_ref, e_ref, w1c_ref, b1_ref, w2_ref, b2_ref,
             ws1_ref, bs1_ref, ws2_ref, bs2_ref,
             wr1_ref, br1_ref, wr2_ref, br2_ref,
             was1_ref, bas1_ref, was2_ref, bas2_ref,
             war1_ref, bar1_ref, war2_ref, bar2_ref,
             emb_ref, ps_ref, pr_ref):
        dot = lambda a, b: jnp.dot(a, b, preferred_element_type=jnp.float32)
        x = gs_ref[...] + gr_ref[...] + dot(e_ref[...], w1c_ref[...]) + b1_ref[...]
        emb = dot(_silu(x), w2_ref[...]) + b2_ref[...]
        emb_ref[...] = emb
        ms = dot(_silu(dot(emb, ws1_ref[...]) + bs1_ref[...]), ws2_ref[...]) + bs2_ref[...]
        mr = dot(_silu(dot(emb, wr1_ref[...]) + br1_ref[...]), wr2_ref[...]) + br2_ref[...]
        has = _silu(dot(emb, was1_ref[...]) + bas1_ref[...])
        har = _silu(dot(emb, war1_ref[...]) + bar1_ref[...])
        ls = jnp.clip(jnp.sum(has * was2_ref[...], axis=1) + bas2_ref[0, 0], -30.0, 30.0)
        lr = jnp.clip(jnp.sum(har * war2_ref[...], axis=1) + bar2_ref[0, 0], -30.0, 30.0)
        p = jnp.exp(ls)[:, None]
        q = jnp.exp(lr)[:, None]
        zpad = jnp.zeros((BE, PW - 65), jnp.float32)
        ps_ref[...] = jnp.concatenate([p * ms, p, zpad], axis=1)
        pr_ref[...] = jnp.concatenate([q * mr, q, zpad], axis=1)

    full = lambda shp: pl.BlockSpec(shp, lambda i: tuple(0 for _ in shp))
    return pl.pallas_call(
        body,
        grid=(N_EDGES // BE,),
        in_specs=[
            pl.BlockSpec((BE, D), lambda i: (i, 0)),
            pl.BlockSpec((BE, D), lambda i: (i, 0)),
            pl.BlockSpec((BE, D), lambda i: (i, 0)),
            full((D, D)), full((1, D)), full((D, D)), full((1, D)),
            full((D, D)), full((1, D)), full((D, 64)), full((1, 64)),
            full((D, D)), full((1, D)), full((D, 64)), full((1, 64)),
            full((D, 32)), full((1, 32)), full((1, 32)), full((1, 1)),
            full((D, 32)), full((1, 32)), full((1, 32)), full((1, 1)),
        ],
        out_specs=[
            pl.BlockSpec((BE, D), lambda i: (i, 0)),
            pl.BlockSpec((BE, PW), lambda i: (i, 0)),
            pl.BlockSpec((BE, PW), lambda i: (i, 0)),
        ],
        out_shape=[
            jax.ShapeDtypeStruct((N_EDGES, D), jnp.float32),
            jax.ShapeDtypeStruct((N_EDGES, PW), jnp.float32),
            jax.ShapeDtypeStruct((N_EDGES, PW), jnp.float32),
        ],
    )(gs, gr, E0, We1c, be1, We2, be2, Ws1, bs1, Ws2, bs2,
      Wr1, br1, Wr2, br2, Was1, bas1, was2, bas2, War1, bar1, war2, bar2)


def _sc_scatter(pay_s, pay_r, sidx, ridx, zrows):
    mesh = plsc.VectorSubcoreMesh(core_axis_name="c", subcore_axis_name="s")

    @functools.partial(
        pl.kernel,
        out_type=jax.ShapeDtypeStruct((NC, N_NODES, PW), jnp.float32),
        mesh=mesh,
        scratch_types=[
            pltpu.VMEM((CH,), jnp.int32),
            pltpu.VMEM((CH, PW), jnp.float32),
            pltpu.VMEM_SHARED((N_NODES, PW), jnp.float32),
            pltpu.SemaphoreType.DMA,
            pltpu.SemaphoreType.DMA,
        ],
    )
    def k(ps_hbm, pr_hbm, sidx_hbm, ridx_hbm, z_hbm, out_hbm,
          idx_v, pay_v, acc, sem1, sem2):
        cid = lax.axis_index("c")
        sid = lax.axis_index("s")
        r0 = pl.multiple_of(sid * ROWS_A, 8)
        t0 = ROWS_A * NS
        pltpu.sync_copy(z_hbm, acc.at[pl.ds(r0, ROWS_A)])

        @pl.when(sid == NS - 1)
        def _():
            pltpu.sync_copy(z_hbm.at[pl.ds(0, ROWS_TAIL)],
                            acc.at[pl.ds(t0, ROWS_TAIL)])

        plsc.subcore_barrier()

        trips = 78 + jnp.where(sid < NCHUNK - 78 * NS, 1, 0)

        def mk_body(idx_hbm, p_hbm):
            def body(kk, carry):
                base = (sid + kk * NS) * CH
                cp1 = pltpu.async_copy(idx_hbm.at[pl.ds(base, CH)], idx_v, sem1)
                cp2 = pltpu.async_copy(p_hbm.at[pl.ds(base, CH)], pay_v, sem2)
                cp1.wait()
                cp2.wait()
                pltpu.sync_copy(pay_v, acc.at[idx_v], add=True)
                return carry
            return body

        @pl.when(cid == 0)
        def _():
            lax.fori_loop(0, trips, mk_body(sidx_hbm, ps_hbm), 0)

        @pl.when(cid == 1)
        def _():
            lax.fori_loop(0, trips, mk_body(ridx_hbm, pr_hbm), 0)

        plsc.subcore_barrier()
        pltpu.sync_copy(acc.at[pl.ds(r0, ROWS_A)],
                        out_hbm.at[cid, pl.ds(r0, ROWS_A)])

        @pl.when(sid == NS - 1)
        def _():
            pltpu.sync_copy(acc.at[pl.ds(t0, ROWS_TAIL)],
                            out_hbm.at[cid, pl.ds(t0, ROWS_TAIL)])

    return k(pay_s, pay_r, sidx, ridx, zrows)


def _node_mlp(V0, parts, Wn1, bn1, Wn2, bn2):
    BN = 1000

    def body(v_ref, parts_ref, wn1_ref, bn1_ref, wn2_ref, bn2_ref, o_ref):
        pr = parts_ref[...]
        ps = pr[0]
        pr_ = pr[1]
        agg0 = ps[:, 0:64] / (ps[:, 64:65] + 1e-30)
        agg1 = pr_[:, 0:64] / (pr_[:, 64:65] + 1e-30)
        ni = jnp.concatenate([v_ref[...], agg0, agg1], axis=1)
        h = _silu(jnp.dot(ni, wn1_ref[...], preferred_element_type=jnp.float32)
                  + bn1_ref[...])
        o_ref[...] = (jnp.dot(h, wn2_ref[...], preferred_element_type=jnp.float32)
                      + bn2_ref[...])

    return pl.pallas_call(
        body,
        grid=(N_NODES // BN,),
        in_specs=[
            pl.BlockSpec((BN, D), lambda i: (i, 0)),
            pl.BlockSpec((NC, BN, PW), lambda i: (0, i, 0)),
            pl.BlockSpec((2 * D, D), lambda i: (0, 0)),
            pl.BlockSpec((1, D), lambda i: (0, 0)),
            pl.BlockSpec((D, D), lambda i: (0, 0)),
            pl.BlockSpec((1, D), lambda i: (0, 0)),
        ],
        out_specs=pl.BlockSpec((BN, D), lambda i: (i, 0)),
        out_shape=jax.ShapeDtypeStruct((N_NODES, D), jnp.float32),
    )(V0, parts, Wn1, bn1, Wn2, bn2)


def kernel(V, E, edges, We1, be1, We2, be2, Ws1, bs1, Ws2, bs2,
           Wr1, br1, Wr2, br2, Was1, bas1, Was2, bas2,
           War1, bar1, War2, bar2, Wn1, bn1, Wn2, bn2):
    V0 = V[0]
    E0 = E[0]
    sidx = edges[0, :, 0]
    ridx = edges[0, :, 1]

    vws, vwr = _pre_project(V0, We1[0:D], We1[D:2 * D])
    gs, gr = _sc_gather(vws, vwr, sidx, ridx)
    emb, pay_s, pay_r = _edge_mlp(
        gs, gr, E0, We1[2 * D:], be1.reshape(1, D), We2, be2.reshape(1, D),
        Ws1, bs1.reshape(1, D), Ws2, bs2.reshape(1, 64),
        Wr1, br1.reshape(1, D), Wr2, br2.reshape(1, 64),
        Was1, bas1.reshape(1, 32), Was2.reshape(1, 32), bas2.reshape(1, 1),
        War1, bar1.reshape(1, 32), War2.reshape(1, 32), bar2.reshape(1, 1))
    zrows = jnp.zeros((ROWS_A, PW), jnp.float32)
    parts = _sc_scatter(pay_s, pay_r, sidx, ridx, zrows)
    node_emb = _node_mlp(V0, parts, Wn1, bn1.reshape(1, D), Wn2,
                         bn2.reshape(1, D))
    return node_emb[None], emb[None]
